# baseline (speedup 1.0000x reference)
import functools

import jax
import jax.numpy as jnp
from jax import lax
from jax.experimental import pallas as pl
from jax.experimental.pallas import tpu as pltpu
from jax.experimental.pallas import tpu_sc as plsc

LANES = 16
BLK = 128
TRI_BLK = 4096


def _flat_idx(n, j):
    return jnp.bitwise_or(
        jnp.bitwise_or(lax.shift_left(jnp.bitwise_and(n, -BLK), 3),
                       lax.shift_left(j, 7)),
        jnp.bitwise_and(n, BLK - 1))


def _tc_build(sent, at_ref, bcol_ref, emb_ref, trip_ref,
              tab_ref, idxs_ref, idxo_ref):
    yt = lax.dot_general(at_ref[...], emb_ref[...], (((1,), (1,)), ((), ())),
                         preferred_element_type=jnp.float32)
    yt = yt + bcol_ref[...]
    u = lax.bitcast_convert_type(yt, jnp.int32)
    carry = jnp.bitwise_and(lax.shift_right_logical(u, 16), 1)
    r16 = lax.shift_right_logical(u + 0x7FFF + carry, 16)
    tab_ref[...] = jnp.bitwise_or(r16[0:8, :], lax.shift_left(r16[8:16, :], 16))

    s = trip_ref[0]
    r = trip_ref[1]
    o = trip_ref[2]
    rc = jnp.minimum(r, 7)
    valid = r < 8
    idxs_ref[...] = jnp.where(valid, _flat_idx(s, rc), sent)
    idxo_ref[...] = jnp.where(valid, _flat_idx(o, rc), sent)


def _sc_body(chunk, sent, tab_hbm, idxs_hbm, idxo_hbm, out_hbm,
             tab_v, is_v, io_v, res_v, sem):
    wid = lax.axis_index("s") * 2 + lax.axis_index("c")
    base = wid * chunk
    tab_cp = pltpu.async_copy(tab_hbm, tab_v, sem)
    pltpu.sync_copy(idxs_hbm.at[pl.ds(base, chunk)], is_v)
    pltpu.sync_copy(idxo_hbm.at[pl.ds(base, chunk)], io_v)
    tab_cp.wait()
    tab_v[pl.ds(sent, LANES)] = jnp.zeros((LANES,), jnp.int32)

    @plsc.parallel_loop(0, chunk, step=LANES, unroll=8)
    def body(i):
        sl = pl.ds(i, LANES)
        gs = plsc.load_gather(tab_v, [is_v[sl]])
        go = plsc.load_gather(tab_v, [io_v[sl]])
        vs = plsc.bitcast(lax.shift_left(gs, 16), jnp.float32)
        vo = plsc.bitcast(jnp.bitwise_and(go, jnp.int32(-65536)), jnp.float32)
        res_v[sl] = vs + vo

    pltpu.sync_copy(res_v, out_hbm.at[pl.ds(base, chunk)])


def kernel(embeddings, triples, W, b):
    n_nodes, n_dim = embeddings.shape
    n_rel = W.shape[0]
    n_triples = triples.shape[1]

    at = jnp.concatenate([W[:, :n_dim], W[:, n_dim:]], axis=0)
    bcol = jnp.concatenate([b, jnp.zeros((n_rel,), jnp.float32)])
    bcol = bcol.reshape(2 * n_rel, 1)

    sent = ((n_nodes & -BLK) << 3) | (n_nodes & (BLK - 1))
    grid = pl.cdiv(n_nodes + 1, BLK)
    assert grid * TRI_BLK >= n_triples
    tab, idxs, idxo = pl.pallas_call(
        functools.partial(_tc_build, sent),
        grid=(grid,),
        in_specs=[
            pl.BlockSpec((2 * n_rel, n_dim), lambda g: (0, 0)),
            pl.BlockSpec((2 * n_rel, 1), lambda g: (0, 0)),
            pl.BlockSpec((BLK, n_dim), lambda g: (g, 0)),
            pl.BlockSpec((3, TRI_BLK), lambda g: (0, g)),
        ],
        out_specs=[
            pl.BlockSpec((n_rel, BLK), lambda g: (g, 0)),
            pl.BlockSpec((TRI_BLK,), lambda g: (g,)),
            pl.BlockSpec((TRI_BLK,), lambda g: (g,)),
        ],
        out_shape=[
            jax.ShapeDtypeStruct((grid * n_rel, BLK), jnp.int32),
            jax.ShapeDtypeStruct((n_triples,), jnp.int32),
            jax.ShapeDtypeStruct((n_triples,), jnp.int32),
        ],
    )(at, bcol, embeddings, triples)

    nw = 32
    chunk = n_triples // nw
    tab_words = grid * n_rel * BLK

    sc = functools.partial(
        pl.kernel,
        mesh=plsc.VectorSubcoreMesh(core_axis_name="c", subcore_axis_name="s"),
        compiler_params=pltpu.CompilerParams(needs_layout_passes=False),
        out_type=jax.ShapeDtypeStruct((n_triples,), jnp.float32),
        scratch_types=[
            pltpu.VMEM((tab_words,), jnp.int32),
            pltpu.VMEM((chunk,), jnp.int32),
            pltpu.VMEM((chunk,), jnp.int32),
            pltpu.VMEM((chunk,), jnp.float32),
            pltpu.SemaphoreType.DMA,
        ],
    )(functools.partial(_sc_body, chunk, sent))

    return sc(tab.reshape(-1), idxs, idxo)


# speedup vs baseline: 1.8512x; 1.8512x over previous
import functools

import jax
import jax.numpy as jnp
from jax import lax
from jax.experimental import pallas as pl
from jax.experimental.pallas import tpu as pltpu
from jax.experimental.pallas import tpu_sc as plsc

LANES = 16
BLK = 128
NBLK = 1024
TRI_BLK = 32768


def _flat_idx(n, j):
    return jnp.bitwise_or(
        jnp.bitwise_or(lax.shift_left(jnp.bitwise_and(n, -BLK), 3),
                       lax.shift_left(j, 7)),
        jnp.bitwise_and(n, BLK - 1))


def _tc_build(sent, at_ref, bcol_ref, emb_ref, trip_ref,
              tab_ref, idxs_ref, idxo_ref):
    yt = lax.dot_general(at_ref[...], emb_ref[...], (((1,), (1,)), ((), ())),
                         preferred_element_type=jnp.float32)
    yt = yt + bcol_ref[...]
    u = lax.bitcast_convert_type(yt, jnp.int32)
    carry = jnp.bitwise_and(lax.shift_right_logical(u, 16), 1)
    r16 = lax.shift_right_logical(u + 0x7FFF + carry, 16)
    packed = jnp.bitwise_or(r16[0:8, :], lax.shift_left(r16[8:16, :], 16))
    for t in range(NBLK // BLK):
        tab_ref[8 * t:8 * t + 8, :] = packed[:, BLK * t:BLK * (t + 1)]

    s = trip_ref[0]
    r = trip_ref[1]
    o = trip_ref[2]
    rc = jnp.minimum(r, 7)
    valid = r < 8
    idxs_ref[...] = jnp.where(valid, _flat_idx(s, rc), sent)
    idxo_ref[...] = jnp.where(valid, _flat_idx(o, rc), sent)


def _sc_body(chunk, sent, tab_hbm, idxs_hbm, idxo_hbm, out_hbm,
             tab_v, is_v, io_v, res_v, sem):
    wid = lax.axis_index("s") * 2 + lax.axis_index("c")
    base = wid * chunk
    tab_cp = pltpu.async_copy(tab_hbm, tab_v, sem)
    pltpu.sync_copy(idxs_hbm.at[pl.ds(base, chunk)], is_v)
    pltpu.sync_copy(idxo_hbm.at[pl.ds(base, chunk)], io_v)
    tab_cp.wait()
    tab_v[pl.ds(sent, LANES)] = jnp.zeros((LANES,), jnp.int32)

    @plsc.parallel_loop(0, chunk, step=LANES, unroll=8)
    def body(i):
        sl = pl.ds(i, LANES)
        gs = plsc.load_gather(tab_v, [is_v[sl]])
        go = plsc.load_gather(tab_v, [io_v[sl]])
        vs = plsc.bitcast(lax.shift_left(gs, 16), jnp.float32)
        vo = plsc.bitcast(jnp.bitwise_and(go, jnp.int32(-65536)), jnp.float32)
        res_v[sl] = vs + vo

    pltpu.sync_copy(res_v, out_hbm.at[pl.ds(base, chunk)])


def kernel(embeddings, triples, W, b):
    n_nodes, n_dim = embeddings.shape
    n_rel = W.shape[0]
    n_triples = triples.shape[1]

    at = jnp.concatenate([W[:, :n_dim], W[:, n_dim:]], axis=0)
    bcol = jnp.concatenate([b, jnp.zeros((n_rel,), jnp.float32)])
    bcol = bcol.reshape(2 * n_rel, 1)

    sent = ((n_nodes & -BLK) << 3) | (n_nodes & (BLK - 1))
    grid = pl.cdiv(n_nodes + 1, NBLK)
    sub = NBLK // BLK
    assert grid * TRI_BLK >= n_triples
    tab, idxs, idxo = pl.pallas_call(
        functools.partial(_tc_build, sent),
        grid=(grid,),
        in_specs=[
            pl.BlockSpec((2 * n_rel, n_dim), lambda g: (0, 0)),
            pl.BlockSpec((2 * n_rel, 1), lambda g: (0, 0)),
            pl.BlockSpec((NBLK, n_dim), lambda g: (g, 0)),
            pl.BlockSpec((3, TRI_BLK), lambda g: (0, g)),
        ],
        out_specs=[
            pl.BlockSpec((sub * n_rel, BLK), lambda g: (g, 0)),
            pl.BlockSpec((TRI_BLK,), lambda g: (g,)),
            pl.BlockSpec((TRI_BLK,), lambda g: (g,)),
        ],
        out_shape=[
            jax.ShapeDtypeStruct((grid * sub * n_rel, BLK), jnp.int32),
            jax.ShapeDtypeStruct((n_triples,), jnp.int32),
            jax.ShapeDtypeStruct((n_triples,), jnp.int32),
        ],
    )(at, bcol, embeddings, triples)

    nw = 32
    chunk = n_triples // nw
    tab_words = grid * sub * n_rel * BLK

    sc = functools.partial(
        pl.kernel,
        mesh=plsc.VectorSubcoreMesh(core_axis_name="c", subcore_axis_name="s"),
        compiler_params=pltpu.CompilerParams(needs_layout_passes=False),
        out_type=jax.ShapeDtypeStruct((n_triples,), jnp.float32),
        scratch_types=[
            pltpu.VMEM((tab_words,), jnp.int32),
            pltpu.VMEM((chunk,), jnp.int32),
            pltpu.VMEM((chunk,), jnp.int32),
            pltpu.VMEM((chunk,), jnp.float32),
            pltpu.SemaphoreType.DMA,
        ],
    )(functools.partial(_sc_body, chunk, sent))

    return sc(tab.reshape(-1), idxs, idxo)


# speedup vs baseline: 2.0421x; 1.1031x over previous
import functools

import jax
import jax.numpy as jnp
from jax import lax
from jax.experimental import pallas as pl
from jax.experimental.pallas import tpu as pltpu
from jax.experimental.pallas import tpu_sc as plsc

LANES = 16
BLK = 128
NBLK = 2048
TRI_BLK = 65536


def _flat_idx(n, j):
    return jnp.bitwise_or(
        jnp.bitwise_or(lax.shift_left(jnp.bitwise_and(n, -BLK), 3),
                       lax.shift_left(j, 7)),
        jnp.bitwise_and(n, BLK - 1))


def _tc_build(sent, n_dim, w_ref, bcol_ref, emb_ref, trip_ref,
              tab_ref, idxs_ref, idxo_ref):
    at = jnp.concatenate([w_ref[:, :n_dim], w_ref[:, n_dim:]], axis=0)
    yt = lax.dot_general(at, emb_ref[...], (((1,), (1,)), ((), ())),
                         preferred_element_type=jnp.float32)
    yt = yt + bcol_ref[...]
    u = lax.bitcast_convert_type(yt, jnp.int32)
    carry = jnp.bitwise_and(lax.shift_right_logical(u, 16), 1)
    r16 = lax.shift_right_logical(u + 0x7FFF + carry, 16)
    packed = jnp.bitwise_or(r16[0:8, :], lax.shift_left(r16[8:16, :], 16))
    for t in range(NBLK // BLK):
        tab_ref[8 * t:8 * t + 8, :] = packed[:, BLK * t:BLK * (t + 1)]

    s = trip_ref[0]
    r = trip_ref[1]
    o = trip_ref[2]
    rc = jnp.minimum(r, 7)
    valid = r < 8
    idxs_ref[...] = jnp.where(valid, _flat_idx(s, rc), sent)
    idxo_ref[...] = jnp.where(valid, _flat_idx(o, rc), sent)


def _sc_body(chunk, sent, tab_hbm, idxs_hbm, idxo_hbm, out_hbm,
             tab_v, is_v, io_v, res_v, sem):
    wid = lax.axis_index("s") * 2 + lax.axis_index("c")
    base = wid * chunk
    tab_cp = pltpu.async_copy(tab_hbm, tab_v, sem)
    pltpu.sync_copy(idxs_hbm.at[pl.ds(base, chunk)], is_v)
    pltpu.sync_copy(idxo_hbm.at[pl.ds(base, chunk)], io_v)
    tab_cp.wait()
    tab_v[pl.ds(sent, LANES)] = jnp.zeros((LANES,), jnp.int32)

    @plsc.parallel_loop(0, chunk, step=LANES, unroll=16)
    def body(i):
        sl = pl.ds(i, LANES)
        gs = plsc.load_gather(tab_v, [is_v[sl]])
        go = plsc.load_gather(tab_v, [io_v[sl]])
        vs = plsc.bitcast(lax.shift_left(gs, 16), jnp.float32)
        vo = plsc.bitcast(jnp.bitwise_and(go, jnp.int32(-65536)), jnp.float32)
        res_v[sl] = vs + vo

    pltpu.sync_copy(res_v, out_hbm.at[pl.ds(base, chunk)])


def kernel(embeddings, triples, W, b):
    n_nodes, n_dim = embeddings.shape
    n_rel = W.shape[0]
    n_triples = triples.shape[1]

    bcol = jnp.concatenate([b, jnp.zeros((n_rel,), jnp.float32)])
    bcol = bcol.reshape(2 * n_rel, 1)

    sent = ((n_nodes & -BLK) << 3) | (n_nodes & (BLK - 1))
    grid = pl.cdiv(n_nodes + 1, NBLK)
    sub = NBLK // BLK
    assert grid * TRI_BLK >= n_triples
    tab, idxs, idxo = pl.pallas_call(
        functools.partial(_tc_build, sent, n_dim),
        grid=(grid,),
        in_specs=[
            pl.BlockSpec((n_rel, 2 * n_dim), lambda g: (0, 0)),
            pl.BlockSpec((2 * n_rel, 1), lambda g: (0, 0)),
            pl.BlockSpec((NBLK, n_dim), lambda g: (g, 0)),
            pl.BlockSpec((3, TRI_BLK), lambda g: (0, g)),
        ],
        out_specs=[
            pl.BlockSpec((sub * n_rel, BLK), lambda g: (g, 0)),
            pl.BlockSpec((TRI_BLK,), lambda g: (g,)),
            pl.BlockSpec((TRI_BLK,), lambda g: (g,)),
        ],
        out_shape=[
            jax.ShapeDtypeStruct((grid * sub * n_rel, BLK), jnp.int32),
            jax.ShapeDtypeStruct((n_triples,), jnp.int32),
            jax.ShapeDtypeStruct((n_triples,), jnp.int32),
        ],
    )(W, bcol, embeddings, triples)

    nw = 32
    chunk = n_triples // nw
    tab_words = grid * sub * n_rel * BLK

    sc = functools.partial(
        pl.kernel,
        mesh=plsc.VectorSubcoreMesh(core_axis_name="c", subcore_axis_name="s"),
        compiler_params=pltpu.CompilerParams(needs_layout_passes=False),
        out_type=jax.ShapeDtypeStruct((n_triples,), jnp.float32),
        scratch_types=[
            pltpu.VMEM((tab_words,), jnp.int32),
            pltpu.VMEM((chunk,), jnp.int32),
            pltpu.VMEM((chunk,), jnp.int32),
            pltpu.VMEM((chunk,), jnp.float32),
            pltpu.SemaphoreType.DMA,
        ],
    )(functools.partial(_sc_body, chunk, sent))

    return sc(tab.reshape(-1), idxs, idxo)


# speedup vs baseline: 2.0478x; 1.0028x over previous
import functools

import jax
import jax.numpy as jnp
from jax import lax
from jax.experimental import pallas as pl
from jax.experimental.pallas import tpu as pltpu
from jax.experimental.pallas import tpu_sc as plsc

LANES = 16
BLK = 128
NBLK = 2048
TRI_BLK = 65536


def _flat_idx(n, j):
    return jnp.bitwise_or(
        jnp.bitwise_or(lax.shift_left(jnp.bitwise_and(n, -BLK), 3),
                       lax.shift_left(j, 7)),
        jnp.bitwise_and(n, BLK - 1))


def _tc_build(sent, n_dim, w_ref, bcol_ref, emb_ref, trip_ref,
              tab_ref, idxs_ref, idxo_ref):
    at = jnp.concatenate([w_ref[:, :n_dim], w_ref[:, n_dim:]], axis=0)
    yt = lax.dot_general(at, emb_ref[...], (((1,), (1,)), ((), ())),
                         preferred_element_type=jnp.float32)
    yt = yt + bcol_ref[...]
    u = lax.bitcast_convert_type(yt, jnp.int32)
    carry = jnp.bitwise_and(lax.shift_right_logical(u, 16), 1)
    r16 = lax.shift_right_logical(u + 0x7FFF + carry, 16)
    packed = jnp.bitwise_or(r16[0:8, :], lax.shift_left(r16[8:16, :], 16))
    for t in range(NBLK // BLK):
        tab_ref[8 * t:8 * t + 8, :] = packed[:, BLK * t:BLK * (t + 1)]

    s = trip_ref[0]
    r = trip_ref[1]
    o = trip_ref[2]
    rc = jnp.minimum(r, 7)
    valid = r < 8
    idxs_ref[...] = jnp.where(valid, _flat_idx(s, rc), sent)
    idxo_ref[...] = jnp.where(valid, _flat_idx(o, rc), sent)


TAB_STREAMS = 4


def _sc_body(chunk, sent, tab_words, tab_hbm, idxs_hbm, idxo_hbm, out_hbm,
             tab_v, is_v, io_v, res_v, sem, sem2):
    wid = lax.axis_index("s") * 2 + lax.axis_index("c")
    base = wid * chunk
    q = tab_words // TAB_STREAMS
    cps = [
        pltpu.async_copy(tab_hbm.at[pl.ds(t * q, q)], tab_v.at[pl.ds(t * q, q)],
                         sem)
        for t in range(TAB_STREAMS)
    ]
    ci = pltpu.async_copy(idxs_hbm.at[pl.ds(base, chunk)], is_v, sem2)
    co = pltpu.async_copy(idxo_hbm.at[pl.ds(base, chunk)], io_v, sem2)
    for cp in cps:
        cp.wait()
    ci.wait()
    co.wait()
    tab_v[pl.ds(sent, LANES)] = jnp.zeros((LANES,), jnp.int32)

    @plsc.parallel_loop(0, chunk, step=LANES, unroll=16)
    def body(i):
        sl = pl.ds(i, LANES)
        gs = plsc.load_gather(tab_v, [is_v[sl]])
        go = plsc.load_gather(tab_v, [io_v[sl]])
        vs = plsc.bitcast(lax.shift_left(gs, 16), jnp.float32)
        vo = plsc.bitcast(jnp.bitwise_and(go, jnp.int32(-65536)), jnp.float32)
        res_v[sl] = vs + vo

    pltpu.sync_copy(res_v, out_hbm.at[pl.ds(base, chunk)])


def kernel(embeddings, triples, W, b):
    n_nodes, n_dim = embeddings.shape
    n_rel = W.shape[0]
    n_triples = triples.shape[1]

    bcol = jnp.concatenate([b, jnp.zeros((n_rel,), jnp.float32)])
    bcol = bcol.reshape(2 * n_rel, 1)

    sent = ((n_nodes & -BLK) << 3) | (n_nodes & (BLK - 1))
    grid = pl.cdiv(n_nodes + 1, NBLK)
    sub = NBLK // BLK
    assert grid * TRI_BLK >= n_triples
    tab, idxs, idxo = pl.pallas_call(
        functools.partial(_tc_build, sent, n_dim),
        grid=(grid,),
        in_specs=[
            pl.BlockSpec((n_rel, 2 * n_dim), lambda g: (0, 0)),
            pl.BlockSpec((2 * n_rel, 1), lambda g: (0, 0)),
            pl.BlockSpec((NBLK, n_dim), lambda g: (g, 0)),
            pl.BlockSpec((3, TRI_BLK), lambda g: (0, g)),
        ],
        out_specs=[
            pl.BlockSpec((sub * n_rel, BLK), lambda g: (g, 0)),
            pl.BlockSpec((TRI_BLK,), lambda g: (g,)),
            pl.BlockSpec((TRI_BLK,), lambda g: (g,)),
        ],
        out_shape=[
            jax.ShapeDtypeStruct((grid * sub * n_rel, BLK), jnp.int32),
            jax.ShapeDtypeStruct((n_triples,), jnp.int32),
            jax.ShapeDtypeStruct((n_triples,), jnp.int32),
        ],
    )(W, bcol, embeddings, triples)

    nw = 32
    chunk = n_triples // nw
    tab_words = grid * sub * n_rel * BLK

    sc = functools.partial(
        pl.kernel,
        mesh=plsc.VectorSubcoreMesh(core_axis_name="c", subcore_axis_name="s"),
        compiler_params=pltpu.CompilerParams(needs_layout_passes=False),
        out_type=jax.ShapeDtypeStruct((n_triples,), jnp.float32),
        scratch_types=[
            pltpu.VMEM((tab_words,), jnp.int32),
            pltpu.VMEM((chunk,), jnp.int32),
            pltpu.VMEM((chunk,), jnp.int32),
            pltpu.VMEM((chunk,), jnp.float32),
            pltpu.SemaphoreType.DMA,
            pltpu.SemaphoreType.DMA,
        ],
    )(functools.partial(_sc_body, chunk, sent, tab_words))

    return sc(tab.reshape(-1), idxs, idxo)


# speedup vs baseline: 2.4229x; 1.1832x over previous
import functools

import jax
import jax.numpy as jnp
from jax import lax
from jax.experimental import pallas as pl
from jax.experimental.pallas import tpu as pltpu
from jax.experimental.pallas import tpu_sc as plsc

LANES = 16
BLK = 128
NBLK = 2048
TRI_BLK = 65536


def _flat_idx(n, j):
    return jnp.bitwise_or(
        jnp.bitwise_or(lax.shift_left(jnp.bitwise_and(n, -BLK), 3),
                       lax.shift_left(j, 7)),
        jnp.bitwise_and(n, BLK - 1))


def _tc_build(sent, n_dim, w_ref, bcol_ref, emb_ref, trip_ref,
              tab_ref, idxs_ref, idxo_ref):
    at = jnp.concatenate([w_ref[:, :n_dim], w_ref[:, n_dim:]], axis=0)
    yt = lax.dot_general(at, emb_ref[...], (((1,), (1,)), ((), ())),
                         preferred_element_type=jnp.float32)
    yt = yt + bcol_ref[...]
    u = lax.bitcast_convert_type(yt, jnp.int32)
    carry = jnp.bitwise_and(lax.shift_right_logical(u, 16), 1)
    r16 = lax.shift_right_logical(u + 0x7FFF + carry, 16)
    packed = jnp.bitwise_or(r16[0:8, :], lax.shift_left(r16[8:16, :], 16))
    for t in range(NBLK // BLK):
        tab_ref[8 * t:8 * t + 8, :] = packed[:, BLK * t:BLK * (t + 1)]

    s = trip_ref[0]
    r = trip_ref[1]
    o = trip_ref[2]
    rc = jnp.minimum(r, 7)
    valid = r < 8
    idxs_ref[...] = jnp.where(valid, _flat_idx(s, rc), sent)
    idxo_ref[...] = jnp.where(valid, _flat_idx(o, rc), sent)


def _sc_body(chunk, sent, tab_words, tab_hbm, idxs_hbm, idxo_hbm, out_hbm,
             tab_sp, tab_v, is_v, io_v, res_v, sem, sem2):
    sid = lax.axis_index("s")
    wid = sid * 2 + lax.axis_index("c")
    base = wid * chunk
    @pl.when(sid == 0)
    def _():
        pltpu.sync_copy(tab_hbm, tab_sp)

    ci = pltpu.async_copy(idxs_hbm.at[pl.ds(base, chunk)], is_v, sem2)
    co = pltpu.async_copy(idxo_hbm.at[pl.ds(base, chunk)], io_v, sem2)
    plsc.subcore_barrier()
    pltpu.sync_copy(tab_sp, tab_v)
    ci.wait()
    co.wait()
    tab_v[pl.ds(sent, LANES)] = jnp.zeros((LANES,), jnp.int32)

    @plsc.parallel_loop(0, chunk, step=LANES, unroll=16)
    def body(i):
        sl = pl.ds(i, LANES)
        gs = plsc.load_gather(tab_v, [is_v[sl]])
        go = plsc.load_gather(tab_v, [io_v[sl]])
        vs = plsc.bitcast(lax.shift_left(gs, 16), jnp.float32)
        vo = plsc.bitcast(jnp.bitwise_and(go, jnp.int32(-65536)), jnp.float32)
        res_v[sl] = vs + vo

    pltpu.sync_copy(res_v, out_hbm.at[pl.ds(base, chunk)])


def kernel(embeddings, triples, W, b):
    n_nodes, n_dim = embeddings.shape
    n_rel = W.shape[0]
    n_triples = triples.shape[1]

    bcol = jnp.concatenate([b, jnp.zeros((n_rel,), jnp.float32)])
    bcol = bcol.reshape(2 * n_rel, 1)

    sent = ((n_nodes & -BLK) << 3) | (n_nodes & (BLK - 1))
    grid = pl.cdiv(n_nodes + 1, NBLK)
    sub = NBLK // BLK
    assert grid * TRI_BLK >= n_triples
    tab, idxs, idxo = pl.pallas_call(
        functools.partial(_tc_build, sent, n_dim),
        grid=(grid,),
        in_specs=[
            pl.BlockSpec((n_rel, 2 * n_dim), lambda g: (0, 0)),
            pl.BlockSpec((2 * n_rel, 1), lambda g: (0, 0)),
            pl.BlockSpec((NBLK, n_dim), lambda g: (g, 0)),
            pl.BlockSpec((3, TRI_BLK), lambda g: (0, g)),
        ],
        out_specs=[
            pl.BlockSpec((sub * n_rel, BLK), lambda g: (g, 0)),
            pl.BlockSpec((TRI_BLK,), lambda g: (g,)),
            pl.BlockSpec((TRI_BLK,), lambda g: (g,)),
        ],
        out_shape=[
            jax.ShapeDtypeStruct((grid * sub * n_rel, BLK), jnp.int32),
            jax.ShapeDtypeStruct((n_triples,), jnp.int32),
            jax.ShapeDtypeStruct((n_triples,), jnp.int32),
        ],
    )(W, bcol, embeddings, triples)

    nw = 32
    chunk = n_triples // nw
    tab_words = grid * sub * n_rel * BLK

    sc = functools.partial(
        pl.kernel,
        mesh=plsc.VectorSubcoreMesh(core_axis_name="c", subcore_axis_name="s"),
        compiler_params=pltpu.CompilerParams(needs_layout_passes=False),
        out_type=jax.ShapeDtypeStruct((n_triples,), jnp.float32),
        scratch_types=[
            pltpu.VMEM_SHARED((tab_words,), jnp.int32),
            pltpu.VMEM((tab_words,), jnp.int32),
            pltpu.VMEM((chunk,), jnp.int32),
            pltpu.VMEM((chunk,), jnp.int32),
            pltpu.VMEM((chunk,), jnp.float32),
            pltpu.SemaphoreType.DMA,
            pltpu.SemaphoreType.DMA,
        ],
    )(functools.partial(_sc_body, chunk, sent, tab_words))

    return sc(tab.reshape(-1), idxs, idxo)
